# baseline (speedup 1.0000x reference)
import functools

import jax
import jax.numpy as jnp
from jax import lax
from jax.experimental import pallas as pl
from jax.experimental.pallas import tpu as pltpu
from jax.experimental.pallas import tpu_sc as plsc

N = 10000
E = 320000
D = 128
NC = 2
NS = 16
NW = NC * NS
EPT = E // NW
EPTP = 10240
BC = 128
BCH = 64
NCH = EPTP // BCH
NPAIR = NCH // 2
N_PAD = 10240
RPT = N_PAD // NS
ZROWS = 128

_f32 = jnp.float32


def _scores_body(h_ref, w_ref, a8_ref, b_ref, attb_ref, out_ref):
    vt = lax.dot_general(a8_ref[...], w_ref[...], (((1,), (0,)), ((), ())),
                         preferred_element_type=_f32)
    s = lax.dot_general(vt, h_ref[...], (((1,), (1,)), ((), ())),
                        preferred_element_type=_f32)
    cvec = lax.dot_general(a8_ref[...], b_ref[...], (((1,), (0,)), ((), ())),
                           preferred_element_type=_f32)
    row = lax.broadcasted_iota(jnp.int32, (8, 1), 0)
    cvec = cvec + jnp.where(row == 0, attb_ref[...], 0.0)
    out_ref[...] = s + cvec


def _scores(h, W, a8, b2, attb):
    return pl.pallas_call(
        _scores_body,
        out_shape=jax.ShapeDtypeStruct((8, N), _f32),
    )(h, W, a8, b2, attb)


def _edge_body(s2, srch, dsth, e_out, ms_out, ss_out,
               tabs, tabd, srcv, dstv, ev, statv):
    c = lax.axis_index("c")
    s = lax.axis_index("s")
    wid = s * NC + c
    pltpu.sync_copy(s2.at[0], tabs)
    pltpu.sync_copy(s2.at[1], tabd)
    pltpu.sync_copy(srch.at[wid], srcv)
    pltpu.sync_copy(dsth.at[wid], dstv)

    def score32(g, _):
        w = srcv[0, pl.ds(g * 16, 16)]
        for half, sidx in enumerate((w & 0xFFFF, (w >> 16) & 0xFFFF)):
            off = g * 32 + half * 16
            a = plsc.load_gather(tabs, [sidx])
            bb = plsc.load_gather(tabd, [dstv[0, pl.ds(off, 16)]])
            z = a + bb
            e16 = jnp.maximum(z, 0.01 * z)
            ev[0, pl.ds(off, 16)] = e16
        return 0

    lax.fori_loop(0, EPTP // 32, score32, 0)

    def padfill(i, _):
        ev[0, pl.ds(i * 16, 16)] = jnp.full((16,), -1e30, _f32)
        return 0

    lax.fori_loop(EPT // 16, EPTP // 16, padfill, 0)

    def max16(i, m):
        return jnp.maximum(m, ev[0, pl.ds(i * 16, 16)])

    m = lax.fori_loop(0, EPTP // 16, max16,
                      jnp.full((16,), -jnp.inf, _f32))
    mt = jnp.max(m)
    mv = jnp.full((16,), mt, _f32)

    def sum16(i, acc):
        return acc + jnp.exp(ev[0, pl.ds(i * 16, 16)] - mv)

    sv = lax.fori_loop(0, EPTP // 16, sum16, jnp.zeros((16,), _f32))
    st = jnp.sum(sv)

    pltpu.sync_copy(ev, e_out.at[wid])
    statv[...] = mv
    pltpu.sync_copy(statv, ms_out.at[wid])
    statv[...] = jnp.full((16,), st, _f32)
    pltpu.sync_copy(statv, ss_out.at[wid])


def _edge_scores(s2, src3, dst3):
    mesh = plsc.VectorSubcoreMesh(core_axis_name="c", subcore_axis_name="s")
    fn = pl.kernel(
        _edge_body,
        out_type=[
            jax.ShapeDtypeStruct((NW, 1, EPTP), _f32),
            jax.ShapeDtypeStruct((NW, 16), _f32),
            jax.ShapeDtypeStruct((NW, 16), _f32),
        ],
        mesh=mesh,
        compiler_params=pltpu.CompilerParams(needs_layout_passes=False),
        scratch_types=[
            pltpu.VMEM((N,), _f32),
            pltpu.VMEM((N,), _f32),
            pltpu.VMEM((1, EPTP // 2), jnp.int32),
            pltpu.VMEM((1, EPTP), jnp.int32),
            pltpu.VMEM((1, EPTP), _f32),
            pltpu.VMEM((16,), _f32),
        ],
    )
    return fn(s2, src3, dst3)


def _scatter_body(h, srch, dsth, eh, ms, ss, part,
                  acc, msv, ssv, srcv, uv,
                  didx0, didx1, dpair0, dpair1, epair0, epair1,
                  sidx0, sidx1, rows0, rows1, sbuf0, sbuf1,
                  gs0, gs1, ps0, ps1, ssm0, ssm1):
    c = lax.axis_index("c")
    s = lax.axis_index("s")
    wid = s * NC + c
    rows_bufs = (rows0, rows1)
    sbufs = (sbuf0, sbuf1)
    sidxs = (sidx0, sidx1)
    didxs = (didx0, didx1)
    dpairs = (dpair0, dpair1)
    epairs = (epair0, epair1)
    gsems = (gs0, gs1)
    psems = (ps0, ps1)
    ssems = (ssm0, ssm1)

    pltpu.sync_copy(srch.at[wid], srcv)

    pltpu.sync_copy(ms, msv)
    pltpu.sync_copy(ss, ssv)

    def mred(i, m):
        return jnp.maximum(m, msv[i, :])

    M = lax.fori_loop(0, NW, mred, jnp.full((16,), -jnp.inf, _f32))

    def sred(i, a):
        return a + ssv[i, :] * jnp.exp(msv[i, :] - M)

    S = lax.fori_loop(0, NW, sred, jnp.zeros((16,), _f32))
    invS = 1.0 / S

    def issue_gather(ci, b):
        sidx = sidxs[b]
        for g in range(BCH // 32):
            w = srcv[0, pl.ds(ci * (BCH // 2) + g * 16, 16)]
            sidx[pl.ds(g * 32, 16)] = w & 0xFFFF
            sidx[pl.ds(g * 32 + 16, 16)] = (w >> 16) & 0xFFFF
        pltpu.async_copy(h.at[sidx], rows_bufs[b], gsems[b])

    def issue_pair(p, pb):
        sem = psems[pb]
        pltpu.async_copy(dsth.at[wid, :, pl.ds(p * BC, BC)], dpairs[pb], sem)
        pltpu.async_copy(eh.at[wid, :, pl.ds(p * BC, BC)], epairs[pb], sem)

    def drain_pair(p, pb):
        sem = psems[pb]
        pltpu.make_async_copy(dsth.at[wid, :, pl.ds(p * BC, BC)],
                              dpairs[pb], sem).wait()
        pltpu.make_async_copy(eh.at[wid, :, pl.ds(p * BC, BC)],
                              epairs[pb], sem).wait()

    def zrow(r, _):
        for j in range(D // 16):
            sbuf0[r, pl.ds(j * 16, 16)] = jnp.zeros((16,), _f32)
        return 0

    lax.fori_loop(0, BCH, zrow, 0)
    for k in range(RPT // BCH):
        pltpu.sync_copy(sbuf0, acc.at[pl.ds(s * RPT + k * BCH, BCH)])

    issue_pair(0, 0)
    issue_pair(1, 1)
    issue_gather(0, 0)
    issue_gather(1, 1)
    plsc.subcore_barrier()

    def process(p, pb, q):
        ci = p * 2 + q
        b = q
        rows = rows_bufs[b]
        sbuf = sbufs[b]
        if q == 0:
            drain_pair(p, pb)

        pltpu.make_async_copy(h.at[sidxs[b]], rows, gsems[b]).wait()

        @pl.when(ci >= 2)
        def _():
            pltpu.make_async_copy(sbuf, acc.at[didxs[b]], ssems[b]).wait()

        for g in range(BCH // 16):
            uv[pl.ds(g * 16, 16)] = (
                jnp.exp(epairs[pb][0, pl.ds(q * BCH + g * 16, 16)] - M)
                * invS)

        def rowscale(bi, _2):
            ub = plsc.load_gather(uv, [jnp.full((16,), bi, jnp.int32)])
            for j in range(D // 16):
                sbuf[bi, pl.ds(j * 16, 16)] = rows[bi, pl.ds(j * 16, 16)] * ub
            return 0

        lax.fori_loop(0, BCH, rowscale, 0)

        @pl.when(ci + 2 < NCH)
        def _():
            issue_gather(ci + 2, b)

        didx = didxs[b]
        for g in range(BCH // 16):
            didx[pl.ds(g * 16, 16)] = (
                dpairs[pb][0, pl.ds(q * BCH + g * 16, 16)])
        pltpu.async_copy(sbuf, acc.at[didx], ssems[b], add=True)

    def quad_step(j, _):
        for pb in range(2):
            p = j * 2 + pb
            process(p, pb, 0)
            process(p, pb, 1)

            @pl.when(p + 2 < NPAIR)
            def _():
                issue_pair(p + 2, pb)
        return 0

    lax.fori_loop(0, NPAIR // 2, quad_step, 0)
    for b in range(2):
        pltpu.make_async_copy(sbufs[b], acc.at[didxs[b]], ssems[b]).wait()
    plsc.subcore_barrier()

    for k in range(RPT // ZROWS):
        r0 = s * RPT + k * ZROWS
        pltpu.sync_copy(acc.at[pl.ds(r0, ZROWS)], part.at[c, pl.ds(r0, ZROWS)])


def _scatter(h, src3, dst3, e3, ms, ss):
    mesh = plsc.VectorSubcoreMesh(core_axis_name="c", subcore_axis_name="s")
    fn = pl.kernel(
        _scatter_body,
        out_type=jax.ShapeDtypeStruct((NC, N_PAD, D), _f32),
        mesh=mesh,
        compiler_params=pltpu.CompilerParams(needs_layout_passes=False),
        scratch_types=[
            pltpu.VMEM_SHARED((N_PAD, D), _f32),
            pltpu.VMEM((NW, 16), _f32),
            pltpu.VMEM((NW, 16), _f32),
            pltpu.VMEM((1, EPTP // 2), jnp.int32),
            pltpu.VMEM((BCH,), _f32),
            pltpu.VMEM((BCH,), jnp.int32),
            pltpu.VMEM((BCH,), jnp.int32),
            pltpu.VMEM((1, BC), jnp.int32),
            pltpu.VMEM((1, BC), jnp.int32),
            pltpu.VMEM((1, BC), _f32),
            pltpu.VMEM((1, BC), _f32),
            pltpu.VMEM((BCH,), jnp.int32),
            pltpu.VMEM((BCH,), jnp.int32),
            pltpu.VMEM((BCH, D), _f32),
            pltpu.VMEM((BCH, D), _f32),
            pltpu.VMEM((BCH, D), _f32),
            pltpu.VMEM((BCH, D), _f32),
            pltpu.SemaphoreType.DMA,
            pltpu.SemaphoreType.DMA,
            pltpu.SemaphoreType.DMA,
            pltpu.SemaphoreType.DMA,
            pltpu.SemaphoreType.DMA,
            pltpu.SemaphoreType.DMA,
        ],
    )
    return fn(h, src3, dst3, e3, ms, ss)


def _combine_body(p_ref, o_ref):
    o_ref[...] = jnp.maximum(p_ref[0] + p_ref[1], 0.0)


def _combine(part):
    nb = 10
    rb = N // nb
    return pl.pallas_call(
        _combine_body,
        grid=(nb,),
        in_specs=[pl.BlockSpec((NC, rb, D), lambda i: (0, i, 0))],
        out_specs=pl.BlockSpec((rb, D), lambda i: (i, 0)),
        out_shape=jax.ShapeDtypeStruct((N, D), _f32),
    )(part)


def kernel(h, edge_index, W, b, att_W, att_b):
    src3 = jnp.pad(edge_index[0].reshape(NW, 1, EPT),
                   ((0, 0), (0, 0), (0, EPTP - EPT)))
    dst3 = jnp.pad(edge_index[1].reshape(NW, 1, EPT),
                   ((0, 0), (0, 0), (0, EPTP - EPT)))
    sblk = src3.reshape(NW, 1, EPTP // 32, 2, 16)
    srcp = (sblk[:, :, :, 0, :] | (sblk[:, :, :, 1, :] << 16)).reshape(
        NW, 1, EPTP // 2)
    a2rows = att_W.reshape(2, D)
    a8 = jnp.zeros((8, D), _f32).at[:2].set(a2rows)
    b2 = b.reshape(D, 1)
    attb = att_b.reshape(1, 1)

    s2 = _scores(h, W, a8, b2, attb)
    e3, ms, ss = _edge_scores(s2, srcp, dst3)
    part = _scatter(h, srcp, dst3, e3, ms, ss)
    return _combine(part)


# speedup vs baseline: 1.4601x; 1.4601x over previous
import jax
import jax.numpy as jnp
from jax import lax
from jax.experimental import pallas as pl
from jax.experimental.pallas import tpu as pltpu
from jax.experimental.pallas import tpu_sc as plsc

N = 10000
E = 320000
D = 128
NC = 2
NS = 16
NW = NC * NS
EPT = E // NW
BC = 80
NCHUNK = EPT // BC
N_PAD = 10240
RPT = N_PAD // NS
ZROWS = 128

_f32 = jnp.float32


def _scores_body(h_ref, w_ref, a8_ref, b_ref, attb_ref, out_ref):
    vt = lax.dot_general(a8_ref[...], w_ref[...], (((1,), (0,)), ((), ())),
                         preferred_element_type=_f32)
    s = lax.dot_general(vt, h_ref[...], (((1,), (1,)), ((), ())),
                        preferred_element_type=_f32)
    cvec = lax.dot_general(a8_ref[...], b_ref[...], (((1,), (0,)), ((), ())),
                           preferred_element_type=_f32)
    row = lax.broadcasted_iota(jnp.int32, (8, 1), 0)
    cvec = cvec + jnp.where(row == 0, attb_ref[...], 0.0)
    out_ref[...] = s + cvec


def _scores(h, W, a8, b2, attb):
    return pl.pallas_call(
        _scores_body,
        out_shape=jax.ShapeDtypeStruct((8, N), _f32),
    )(h, W, a8, b2, attb)


def _edge_body(s2, srch, dsth, e_out, ms_out, ss_out,
               tabs, tabd, srcv, dstv, ev, statv):
    c = lax.axis_index("c")
    s = lax.axis_index("s")
    wid = s * NC + c
    base = wid * EPT
    pltpu.sync_copy(s2.at[0], tabs)
    pltpu.sync_copy(s2.at[1], tabd)
    pltpu.sync_copy(srch.at[pl.ds(base, EPT)], srcv)
    pltpu.sync_copy(dsth.at[pl.ds(base, EPT)], dstv)

    def score16(i, m):
        a = plsc.load_gather(tabs, [srcv[pl.ds(i * 16, 16)]])
        bb = plsc.load_gather(tabd, [dstv[pl.ds(i * 16, 16)]])
        z = a + bb
        e16 = jnp.maximum(z, 0.01 * z)
        ev[pl.ds(i * 16, 16)] = e16
        return jnp.maximum(m, e16)

    m = lax.fori_loop(0, EPT // 16, score16,
                      jnp.full((16,), -jnp.inf, _f32))
    mt = jnp.max(m)
    mv = jnp.full((16,), mt, _f32)

    def sum16(i, acc):
        return acc + jnp.exp(ev[pl.ds(i * 16, 16)] - mv)

    sv = lax.fori_loop(0, EPT // 16, sum16, jnp.zeros((16,), _f32))
    st = jnp.sum(sv)

    pltpu.sync_copy(ev, e_out.at[pl.ds(base, EPT)])
    statv[...] = mv
    pltpu.sync_copy(statv, ms_out.at[wid])
    statv[...] = jnp.full((16,), st, _f32)
    pltpu.sync_copy(statv, ss_out.at[wid])


def _edge_scores(s2, src, dst):
    mesh = plsc.VectorSubcoreMesh(core_axis_name="c", subcore_axis_name="s")
    fn = pl.kernel(
        _edge_body,
        out_type=[
            jax.ShapeDtypeStruct((E,), _f32),
            jax.ShapeDtypeStruct((NW, 16), _f32),
            jax.ShapeDtypeStruct((NW, 16), _f32),
        ],
        mesh=mesh,
        compiler_params=pltpu.CompilerParams(needs_layout_passes=False),
        scratch_types=[
            pltpu.VMEM((N,), _f32),
            pltpu.VMEM((N,), _f32),
            pltpu.VMEM((EPT,), jnp.int32),
            pltpu.VMEM((EPT,), jnp.int32),
            pltpu.VMEM((EPT,), _f32),
            pltpu.VMEM((16,), _f32),
        ],
    )
    return fn(s2, src, dst)


def _scatter_body(h, srch, dsth, eh, ms, ss, part,
                  acc, msv, ssv, srcv, dstv, ev, uv, rows, zbuf, sem):
    c = lax.axis_index("c")
    s = lax.axis_index("s")
    wid = s * NC + c
    base = wid * EPT

    pltpu.sync_copy(ms, msv)
    pltpu.sync_copy(ss, ssv)

    def mred(i, m):
        return jnp.maximum(m, msv[i, :])

    M = lax.fori_loop(0, NW, mred, jnp.full((16,), -jnp.inf, _f32))

    def sred(i, a):
        return a + ssv[i, :] * jnp.exp(msv[i, :] - M)

    S = lax.fori_loop(0, NW, sred, jnp.zeros((16,), _f32))
    invS = 1.0 / S

    def zrow(r, _):
        for j in range(D // 16):
            zbuf[r, pl.ds(j * 16, 16)] = jnp.zeros((16,), _f32)
        return 0

    lax.fori_loop(0, ZROWS, zrow, 0)
    for k in range(RPT // ZROWS):
        pltpu.sync_copy(zbuf, acc.at[pl.ds(s * RPT + k * ZROWS, ZROWS)])
    plsc.subcore_barrier()

    def chunk(ci, _):
        off = base + ci * BC
        pltpu.sync_copy(srch.at[pl.ds(off, BC)], srcv)
        pltpu.sync_copy(dsth.at[pl.ds(off, BC)], dstv)
        pltpu.sync_copy(eh.at[pl.ds(off, BC)], ev)
        pltpu.async_copy(h.at[srcv], rows, sem).wait()
        for g in range(BC // 16):
            uv[pl.ds(g * 16, 16)] = (
                jnp.exp(ev[pl.ds(g * 16, 16)] - M) * invS)

        def rowscale(bi, _2):
            ub = plsc.load_gather(uv, [jnp.full((16,), bi, jnp.int32)])
            for j in range(D // 16):
                rows[bi, pl.ds(j * 16, 16)] = rows[bi, pl.ds(j * 16, 16)] * ub
            return 0

        lax.fori_loop(0, BC, rowscale, 0)
        pltpu.sync_copy(rows, acc.at[dstv], add=True)
        return 0

    lax.fori_loop(0, NCHUNK, chunk, 0)
    plsc.subcore_barrier()

    for k in range(RPT // ZROWS):
        r0 = s * RPT + k * ZROWS
        pltpu.sync_copy(acc.at[pl.ds(r0, ZROWS)], part.at[c, pl.ds(r0, ZROWS)])


def _scatter(h, src, dst, e, ms, ss):
    mesh = plsc.VectorSubcoreMesh(core_axis_name="c", subcore_axis_name="s")
    fn = pl.kernel(
        _scatter_body,
        out_type=jax.ShapeDtypeStruct((NC, N_PAD, D), _f32),
        mesh=mesh,
        compiler_params=pltpu.CompilerParams(needs_layout_passes=False),
        scratch_types=[
            pltpu.VMEM_SHARED((N_PAD, D), _f32),
            pltpu.VMEM((NW, 16), _f32),
            pltpu.VMEM((NW, 16), _f32),
            pltpu.VMEM((BC,), jnp.int32),
            pltpu.VMEM((BC,), jnp.int32),
            pltpu.VMEM((BC,), _f32),
            pltpu.VMEM((BC,), _f32),
            pltpu.VMEM((BC, D), _f32),
            pltpu.VMEM((ZROWS, D), _f32),
            pltpu.SemaphoreType.DMA,
        ],
    )
    return fn(h, src, dst, e, ms, ss)


def _combine_body(p_ref, o_ref):
    o_ref[...] = jnp.maximum(p_ref[0] + p_ref[1], 0.0)


def _combine(part):
    nb = 10
    rb = N // nb
    return pl.pallas_call(
        _combine_body,
        grid=(nb,),
        in_specs=[pl.BlockSpec((NC, rb, D), lambda i: (0, i, 0))],
        out_specs=pl.BlockSpec((rb, D), lambda i: (i, 0)),
        out_shape=jax.ShapeDtypeStruct((N, D), _f32),
    )(part)


def kernel(h, edge_index, W, b, att_W, att_b):
    src = edge_index[0]
    dst = edge_index[1]
    a2rows = att_W.reshape(2, D)
    a8 = jnp.zeros((8, D), _f32).at[:2].set(a2rows)
    b2 = b.reshape(D, 1)
    attb = att_b.reshape(1, 1)

    s2 = _scores(h, W, a8, b2, attb)
    e, ms, ss = _edge_scores(s2, src, dst)
    part = _scatter(h, src, dst, e, ms, ss)
    return _combine(part)


# speedup vs baseline: 1.8029x; 1.2348x over previous
import jax
import jax.numpy as jnp
from jax import lax
from jax.experimental import pallas as pl
from jax.experimental.pallas import tpu as pltpu
from jax.experimental.pallas import tpu_sc as plsc

N = 10000
E = 320000
D = 128
NC = 2
NS = 16
NW = NC * NS
EPT = E // NW
BC = 80
NCHUNK = EPT // BC
N_PAD = 10240
RPT = N_PAD // NS
ZROWS = 128

_f32 = jnp.float32


def _scores_body(h_ref, w_ref, a8_ref, b_ref, attb_ref, out_ref):
    vt = lax.dot_general(a8_ref[...], w_ref[...], (((1,), (0,)), ((), ())),
                         preferred_element_type=_f32)
    s = lax.dot_general(vt, h_ref[...], (((1,), (1,)), ((), ())),
                        preferred_element_type=_f32)
    cvec = lax.dot_general(a8_ref[...], b_ref[...], (((1,), (0,)), ((), ())),
                           preferred_element_type=_f32)
    row = lax.broadcasted_iota(jnp.int32, (8, 1), 0)
    cvec = cvec + jnp.where(row == 0, attb_ref[...], 0.0)
    out_ref[...] = s + cvec


def _scores(h, W, a8, b2, attb):
    return pl.pallas_call(
        _scores_body,
        out_shape=jax.ShapeDtypeStruct((8, N), _f32),
    )(h, W, a8, b2, attb)


def _edge_body(s2, srch, dsth, e_out, ms_out, ss_out,
               tabs, tabd, srcv, dstv, ev, statv):
    c = lax.axis_index("c")
    s = lax.axis_index("s")
    wid = s * NC + c
    base = wid * EPT
    pltpu.sync_copy(s2.at[0], tabs)
    pltpu.sync_copy(s2.at[1], tabd)
    pltpu.sync_copy(srch.at[pl.ds(base, EPT)], srcv)
    pltpu.sync_copy(dsth.at[pl.ds(base, EPT)], dstv)

    def score16(i, m):
        a = plsc.load_gather(tabs, [srcv[pl.ds(i * 16, 16)]])
        bb = plsc.load_gather(tabd, [dstv[pl.ds(i * 16, 16)]])
        z = a + bb
        e16 = jnp.maximum(z, 0.01 * z)
        ev[pl.ds(i * 16, 16)] = e16
        return jnp.maximum(m, e16)

    m = lax.fori_loop(0, EPT // 16, score16,
                      jnp.full((16,), -jnp.inf, _f32))
    mt = jnp.max(m)
    mv = jnp.full((16,), mt, _f32)

    def sum16(i, acc):
        return acc + jnp.exp(ev[pl.ds(i * 16, 16)] - mv)

    sv = lax.fori_loop(0, EPT // 16, sum16, jnp.zeros((16,), _f32))
    st = jnp.sum(sv)

    pltpu.sync_copy(ev, e_out.at[pl.ds(base, EPT)])
    statv[...] = mv
    pltpu.sync_copy(statv, ms_out.at[wid])
    statv[...] = jnp.full((16,), st, _f32)
    pltpu.sync_copy(statv, ss_out.at[wid])


def _edge_scores(s2, src, dst):
    mesh = plsc.VectorSubcoreMesh(core_axis_name="c", subcore_axis_name="s")
    fn = pl.kernel(
        _edge_body,
        out_type=[
            jax.ShapeDtypeStruct((E,), _f32),
            jax.ShapeDtypeStruct((NW, 16), _f32),
            jax.ShapeDtypeStruct((NW, 16), _f32),
        ],
        mesh=mesh,
        compiler_params=pltpu.CompilerParams(needs_layout_passes=False),
        scratch_types=[
            pltpu.VMEM((N,), _f32),
            pltpu.VMEM((N,), _f32),
            pltpu.VMEM((EPT,), jnp.int32),
            pltpu.VMEM((EPT,), jnp.int32),
            pltpu.VMEM((EPT,), _f32),
            pltpu.VMEM((16,), _f32),
        ],
    )
    return fn(s2, src, dst)


def _scatter_body(h, srch, dsth, eh, ms, ss, part,
                  acc, msv, ssv, srcv, dstv, ev, uv, rows, zbuf, sem):
    c = lax.axis_index("c")
    s = lax.axis_index("s")
    wid = s * NC + c
    base = wid * EPT

    pltpu.sync_copy(ms, msv)
    pltpu.sync_copy(ss, ssv)

    def mred(i, m):
        return jnp.maximum(m, msv[i, :])

    M = lax.fori_loop(0, NW, mred, jnp.full((16,), -jnp.inf, _f32))

    def sred(i, a):
        return a + ssv[i, :] * jnp.exp(msv[i, :] - M)

    S = lax.fori_loop(0, NW, sred, jnp.zeros((16,), _f32))
    invS = 1.0 / S

    def zrow(r, _):
        for j in range(D // 16):
            zbuf[r, pl.ds(j * 16, 16)] = jnp.zeros((16,), _f32)
        return 0

    lax.fori_loop(0, ZROWS, zrow, 0)
    for k in range(RPT // ZROWS):
        pltpu.sync_copy(zbuf, acc.at[pl.ds(s * RPT + k * ZROWS, ZROWS)])
    plsc.subcore_barrier()

    def chunk(ci, _):
        off = base + ci * BC
        pltpu.async_copy(srch.at[pl.ds(off, BC)], srcv, sem)
        pltpu.async_copy(dsth.at[pl.ds(off, BC)], dstv, sem)
        pltpu.async_copy(eh.at[pl.ds(off, BC)], ev, sem)
        pltpu.make_async_copy(srch.at[pl.ds(off, BC)], srcv, sem).wait()
        pltpu.make_async_copy(dsth.at[pl.ds(off, BC)], dstv, sem).wait()
        pltpu.make_async_copy(eh.at[pl.ds(off, BC)], ev, sem).wait()
        pltpu.async_copy(h.at[srcv], rows, sem).wait()
        for g in range(BC // 16):
            uv[pl.ds(g * 16, 16)] = (
                jnp.exp(ev[pl.ds(g * 16, 16)] - M) * invS)

        def rowscale(bi, _2):
            ub = plsc.load_gather(uv, [jnp.full((16,), bi, jnp.int32)])
            for j in range(D // 16):
                rows[bi, pl.ds(j * 16, 16)] = rows[bi, pl.ds(j * 16, 16)] * ub
            return 0

        lax.fori_loop(0, BC, rowscale, 0)
        pltpu.sync_copy(rows, acc.at[dstv], add=True)
        return 0

    lax.fori_loop(0, NCHUNK, chunk, 0)
    plsc.subcore_barrier()

    for k in range(RPT // ZROWS):
        r0 = s * RPT + k * ZROWS
        pltpu.sync_copy(acc.at[pl.ds(r0, ZROWS)], part.at[c, pl.ds(r0, ZROWS)])


def _scatter(h, src, dst, e, ms, ss):
    mesh = plsc.VectorSubcoreMesh(core_axis_name="c", subcore_axis_name="s")
    fn = pl.kernel(
        _scatter_body,
        out_type=jax.ShapeDtypeStruct((NC, N_PAD, D), _f32),
        mesh=mesh,
        compiler_params=pltpu.CompilerParams(needs_layout_passes=False),
        scratch_types=[
            pltpu.VMEM_SHARED((N_PAD, D), _f32),
            pltpu.VMEM((NW, 16), _f32),
            pltpu.VMEM((NW, 16), _f32),
            pltpu.VMEM((BC,), jnp.int32),
            pltpu.VMEM((BC,), jnp.int32),
            pltpu.VMEM((BC,), _f32),
            pltpu.VMEM((BC,), _f32),
            pltpu.VMEM((BC, D), _f32),
            pltpu.VMEM((ZROWS, D), _f32),
            pltpu.SemaphoreType.DMA,
        ],
    )
    return fn(h, src, dst, e, ms, ss)


def _combine_body(p_ref, o_ref):
    o_ref[...] = jnp.maximum(p_ref[0] + p_ref[1], 0.0)


def _combine(part):
    nb = 10
    rb = N // nb
    return pl.pallas_call(
        _combine_body,
        grid=(nb,),
        in_specs=[pl.BlockSpec((NC, rb, D), lambda i: (0, i, 0))],
        out_specs=pl.BlockSpec((rb, D), lambda i: (i, 0)),
        out_shape=jax.ShapeDtypeStruct((N, D), _f32),
    )(part)


def kernel(h, edge_index, W, b, att_W, att_b):
    src = edge_index[0]
    dst = edge_index[1]
    a2rows = att_W.reshape(2, D)
    a8 = jnp.zeros((8, D), _f32).at[:2].set(a2rows)
    b2 = b.reshape(D, 1)
    attb = att_b.reshape(1, 1)

    s2 = _scores(h, W, a8, b2, attb)
    e, ms, ss = _edge_scores(s2, src, dst)
    part = _scatter(h, src, dst, e, ms, ss)
    return _combine(part)


# speedup vs baseline: 1.9886x; 1.1030x over previous
import jax
import jax.numpy as jnp
from jax import lax
from jax.experimental import pallas as pl
from jax.experimental.pallas import tpu as pltpu
from jax.experimental.pallas import tpu_sc as plsc

N = 10000
E = 320000
D = 128
NC = 2
NS = 16
NW = NC * NS
EPT = E // NW
BC = 80
NCHUNK = EPT // BC
GB = 5
N_PAD = 10240
RPT = N_PAD // NS
ZROWS = 128

_f32 = jnp.float32


def _scores_body(h_ref, w_ref, a8_ref, b_ref, attb_ref, out_ref):
    vt = lax.dot_general(a8_ref[...], w_ref[...], (((1,), (0,)), ((), ())),
                         preferred_element_type=_f32)
    s = lax.dot_general(vt, h_ref[...], (((1,), (1,)), ((), ())),
                        preferred_element_type=_f32)
    cvec = lax.dot_general(a8_ref[...], b_ref[...], (((1,), (0,)), ((), ())),
                           preferred_element_type=_f32)
    row = lax.broadcasted_iota(jnp.int32, (8, 1), 0)
    cvec = cvec + jnp.where(row == 0, attb_ref[...], 0.0)
    out_ref[...] = s + cvec


def _scores(h, W, a8, b2, attb):
    return pl.pallas_call(
        _scores_body,
        out_shape=jax.ShapeDtypeStruct((8, N), _f32),
    )(h, W, a8, b2, attb)


def _edge_body(s2, srch, dsth, e_out, ms_out, ss_out,
               tabs, tabd, srcv, dstv, ev, statv):
    c = lax.axis_index("c")
    s = lax.axis_index("s")
    wid = s * NC + c
    base = wid * EPT
    pltpu.sync_copy(s2.at[0], tabs)
    pltpu.sync_copy(s2.at[1], tabd)
    pltpu.sync_copy(srch.at[pl.ds(base, EPT)], srcv)
    pltpu.sync_copy(dsth.at[pl.ds(base, EPT)], dstv)

    def score16(i, m):
        a = plsc.load_gather(tabs, [srcv[pl.ds(i * 16, 16)]])
        bb = plsc.load_gather(tabd, [dstv[pl.ds(i * 16, 16)]])
        z = a + bb
        e16 = jnp.maximum(z, 0.01 * z)
        ev[pl.ds(i * 16, 16)] = e16
        return jnp.maximum(m, e16)

    m = lax.fori_loop(0, EPT // 16, score16,
                      jnp.full((16,), -jnp.inf, _f32))
    mt = jnp.max(m)
    mv = jnp.full((16,), mt, _f32)

    def sum16(i, acc):
        return acc + jnp.exp(ev[pl.ds(i * 16, 16)] - mv)

    sv = lax.fori_loop(0, EPT // 16, sum16, jnp.zeros((16,), _f32))
    st = jnp.sum(sv)

    pltpu.sync_copy(ev, e_out.at[pl.ds(base, EPT)])
    statv[...] = mv
    pltpu.sync_copy(statv, ms_out.at[wid])
    statv[...] = jnp.full((16,), st, _f32)
    pltpu.sync_copy(statv, ss_out.at[wid])


def _edge_scores(s2, src, dst):
    mesh = plsc.VectorSubcoreMesh(core_axis_name="c", subcore_axis_name="s")
    fn = pl.kernel(
        _edge_body,
        out_type=[
            jax.ShapeDtypeStruct((E,), _f32),
            jax.ShapeDtypeStruct((NW, 16), _f32),
            jax.ShapeDtypeStruct((NW, 16), _f32),
        ],
        mesh=mesh,
        compiler_params=pltpu.CompilerParams(needs_layout_passes=False),
        scratch_types=[
            pltpu.VMEM((N,), _f32),
            pltpu.VMEM((N,), _f32),
            pltpu.VMEM((EPT,), jnp.int32),
            pltpu.VMEM((EPT,), jnp.int32),
            pltpu.VMEM((EPT,), _f32),
            pltpu.VMEM((16,), _f32),
        ],
    )
    return fn(s2, src, dst)


def _scatter_body(h, srch, dsth, eh, ms, ss, part,
                  acc, msv, ssv, srcv, dstv, ev, uv, sidx, didx, rows, zbuf,
                  sem):
    c = lax.axis_index("c")
    s = lax.axis_index("s")
    wid = s * NC + c
    base = wid * EPT

    pltpu.sync_copy(ms, msv)
    pltpu.sync_copy(ss, ssv)

    def mred(i, m):
        return jnp.maximum(m, msv[i, :])

    M = lax.fori_loop(0, NW, mred, jnp.full((16,), -jnp.inf, _f32))

    def sred(i, a):
        return a + ssv[i, :] * jnp.exp(msv[i, :] - M)

    S = lax.fori_loop(0, NW, sred, jnp.zeros((16,), _f32))
    invS = 1.0 / S

    def zrow(r, _):
        for j in range(D // 16):
            zbuf[r, pl.ds(j * 16, 16)] = jnp.zeros((16,), _f32)
        return 0

    lax.fori_loop(0, ZROWS, zrow, 0)
    for k in range(RPT // ZROWS):
        pltpu.sync_copy(zbuf, acc.at[pl.ds(s * RPT + k * ZROWS, ZROWS)])
    plsc.subcore_barrier()

    def superchunk(si, _):
        off = base + si * (GB * BC)
        pltpu.async_copy(srch.at[pl.ds(off, GB * BC)], srcv, sem)
        pltpu.async_copy(dsth.at[pl.ds(off, GB * BC)], dstv, sem)
        pltpu.async_copy(eh.at[pl.ds(off, GB * BC)], ev, sem)
        pltpu.make_async_copy(srch.at[pl.ds(off, GB * BC)], srcv, sem).wait()
        pltpu.make_async_copy(dsth.at[pl.ds(off, GB * BC)], dstv, sem).wait()
        pltpu.make_async_copy(eh.at[pl.ds(off, GB * BC)], ev, sem).wait()
        for q in range(GB):
            qo = q * BC
            for g in range(BC // 16):
                sidx[pl.ds(g * 16, 16)] = srcv[pl.ds(qo + g * 16, 16)]
            pltpu.async_copy(h.at[sidx], rows, sem).wait()
            for g in range(BC // 16):
                uv[pl.ds(g * 16, 16)] = (
                    jnp.exp(ev[pl.ds(qo + g * 16, 16)] - M) * invS)

            def rowscale(bi, _2):
                ub = plsc.load_gather(uv, [jnp.full((16,), bi, jnp.int32)])
                for j in range(D // 16):
                    rows[bi, pl.ds(j * 16, 16)] = (
                        rows[bi, pl.ds(j * 16, 16)] * ub)
                return 0

            lax.fori_loop(0, BC, rowscale, 0)
            for g in range(BC // 16):
                didx[pl.ds(g * 16, 16)] = dstv[pl.ds(qo + g * 16, 16)]
            pltpu.sync_copy(rows, acc.at[didx], add=True)
        return 0

    lax.fori_loop(0, NCHUNK // GB, superchunk, 0)
    plsc.subcore_barrier()

    for k in range(RPT // ZROWS):
        r0 = s * RPT + k * ZROWS
        pltpu.sync_copy(acc.at[pl.ds(r0, ZROWS)], part.at[c, pl.ds(r0, ZROWS)])


def _scatter(h, src, dst, e, ms, ss):
    mesh = plsc.VectorSubcoreMesh(core_axis_name="c", subcore_axis_name="s")
    fn = pl.kernel(
        _scatter_body,
        out_type=jax.ShapeDtypeStruct((NC, N_PAD, D), _f32),
        mesh=mesh,
        compiler_params=pltpu.CompilerParams(needs_layout_passes=False),
        scratch_types=[
            pltpu.VMEM_SHARED((N_PAD, D), _f32),
            pltpu.VMEM((NW, 16), _f32),
            pltpu.VMEM((NW, 16), _f32),
            pltpu.VMEM((GB * BC,), jnp.int32),
            pltpu.VMEM((GB * BC,), jnp.int32),
            pltpu.VMEM((GB * BC,), _f32),
            pltpu.VMEM((BC,), _f32),
            pltpu.VMEM((BC,), jnp.int32),
            pltpu.VMEM((BC,), jnp.int32),
            pltpu.VMEM((BC, D), _f32),
            pltpu.VMEM((ZROWS, D), _f32),
            pltpu.SemaphoreType.DMA,
        ],
    )
    return fn(h, src, dst, e, ms, ss)


def _combine_body(p_ref, o_ref):
    o_ref[...] = jnp.maximum(p_ref[0] + p_ref[1], 0.0)


def _combine(part):
    nb = 10
    rb = N // nb
    return pl.pallas_call(
        _combine_body,
        grid=(nb,),
        in_specs=[pl.BlockSpec((NC, rb, D), lambda i: (0, i, 0))],
        out_specs=pl.BlockSpec((rb, D), lambda i: (i, 0)),
        out_shape=jax.ShapeDtypeStruct((N, D), _f32),
    )(part)


def kernel(h, edge_index, W, b, att_W, att_b):
    src = edge_index[0]
    dst = edge_index[1]
    a2rows = att_W.reshape(2, D)
    a8 = jnp.zeros((8, D), _f32).at[:2].set(a2rows)
    b2 = b.reshape(D, 1)
    attb = att_b.reshape(1, 1)

    s2 = _scores(h, W, a8, b2, attb)
    e, ms, ss = _edge_scores(s2, src, dst)
    part = _scatter(h, src, dst, e, ms, ss)
    return _combine(part)


# speedup vs baseline: 2.7169x; 1.3663x over previous
import jax
import jax.numpy as jnp
from jax import lax
from jax.experimental import pallas as pl
from jax.experimental.pallas import tpu as pltpu
from jax.experimental.pallas import tpu_sc as plsc

N = 10000
E = 320000
D = 128
NC = 2
NS = 16
NW = NC * NS
EPT = E // NW
BC = 80
NCHUNK = EPT // BC
GB = 5
N_PAD = 10240
RPT = N_PAD // NS
ZROWS = 128

_f32 = jnp.float32


def _scores_body(h_ref, w_ref, a8_ref, b_ref, attb_ref, out_ref):
    vt = lax.dot_general(a8_ref[...], w_ref[...], (((1,), (0,)), ((), ())),
                         preferred_element_type=_f32)
    s = lax.dot_general(vt, h_ref[...], (((1,), (1,)), ((), ())),
                        preferred_element_type=_f32)
    cvec = lax.dot_general(a8_ref[...], b_ref[...], (((1,), (0,)), ((), ())),
                           preferred_element_type=_f32)
    row = lax.broadcasted_iota(jnp.int32, (8, 1), 0)
    cvec = cvec + jnp.where(row == 0, attb_ref[...], 0.0)
    out_ref[...] = s + cvec


def _scores(h, W, a8, b2, attb):
    return pl.pallas_call(
        _scores_body,
        out_shape=jax.ShapeDtypeStruct((8, N), _f32),
    )(h, W, a8, b2, attb)


def _edge_body(s2, srch, dsth, e_out, ms_out, ss_out,
               tabs, tabd, srcv, dstv, ev, statv):
    c = lax.axis_index("c")
    s = lax.axis_index("s")
    wid = s * NC + c
    base = wid * EPT
    pltpu.sync_copy(s2.at[0], tabs)
    pltpu.sync_copy(s2.at[1], tabd)
    pltpu.sync_copy(srch.at[pl.ds(base, EPT)], srcv)
    pltpu.sync_copy(dsth.at[pl.ds(base, EPT)], dstv)

    def score16(i, m):
        a = plsc.load_gather(tabs, [srcv[pl.ds(i * 16, 16)]])
        bb = plsc.load_gather(tabd, [dstv[pl.ds(i * 16, 16)]])
        z = a + bb
        e16 = jnp.maximum(z, 0.01 * z)
        ev[pl.ds(i * 16, 16)] = e16
        return jnp.maximum(m, e16)

    m = lax.fori_loop(0, EPT // 16, score16,
                      jnp.full((16,), -jnp.inf, _f32))
    mt = jnp.max(m)
    mv = jnp.full((16,), mt, _f32)

    def sum16(i, acc):
        return acc + jnp.exp(ev[pl.ds(i * 16, 16)] - mv)

    sv = lax.fori_loop(0, EPT // 16, sum16, jnp.zeros((16,), _f32))
    st = jnp.sum(sv)

    pltpu.sync_copy(ev, e_out.at[pl.ds(base, EPT)])
    statv[...] = mv
    pltpu.sync_copy(statv, ms_out.at[wid])
    statv[...] = jnp.full((16,), st, _f32)
    pltpu.sync_copy(statv, ss_out.at[wid])


def _edge_scores(s2, src, dst):
    mesh = plsc.VectorSubcoreMesh(core_axis_name="c", subcore_axis_name="s")
    fn = pl.kernel(
        _edge_body,
        out_type=[
            jax.ShapeDtypeStruct((E,), _f32),
            jax.ShapeDtypeStruct((NW, 16), _f32),
            jax.ShapeDtypeStruct((NW, 16), _f32),
        ],
        mesh=mesh,
        compiler_params=pltpu.CompilerParams(needs_layout_passes=False),
        scratch_types=[
            pltpu.VMEM((N,), _f32),
            pltpu.VMEM((N,), _f32),
            pltpu.VMEM((EPT,), jnp.int32),
            pltpu.VMEM((EPT,), jnp.int32),
            pltpu.VMEM((EPT,), _f32),
            pltpu.VMEM((16,), _f32),
        ],
    )
    return fn(s2, src, dst)


def _scatter_body(h, srch, dsth, eh, ms, ss, part,
                  acc, msv, ssv, srcv, dstv, ev, uv, sidx0, sidx1, didx,
                  rows0, rows1, zbuf, sem, gs0, gs1):
    sidxs = (sidx0, sidx1)
    rows_bufs = (rows0, rows1)
    gsems = (gs0, gs1)
    c = lax.axis_index("c")
    s = lax.axis_index("s")
    wid = s * NC + c
    base = wid * EPT

    pltpu.sync_copy(ms, msv)
    pltpu.sync_copy(ss, ssv)

    def mred(i, m):
        return jnp.maximum(m, msv[i, :])

    M = lax.fori_loop(0, NW, mred, jnp.full((16,), -jnp.inf, _f32))

    def sred(i, a):
        return a + ssv[i, :] * jnp.exp(msv[i, :] - M)

    S = lax.fori_loop(0, NW, sred, jnp.zeros((16,), _f32))
    invS = 1.0 / S

    def zrow(r, _):
        for j in range(D // 16):
            zbuf[r, pl.ds(j * 16, 16)] = jnp.zeros((16,), _f32)
        return 0

    lax.fori_loop(0, ZROWS, zrow, 0)
    for k in range(RPT // ZROWS):
        pltpu.sync_copy(zbuf, acc.at[pl.ds(s * RPT + k * ZROWS, ZROWS)])
    plsc.subcore_barrier()

    def superchunk(si, _):
        off = base + si * (GB * BC)
        pltpu.async_copy(srch.at[pl.ds(off, GB * BC)], srcv, sem)
        pltpu.async_copy(dsth.at[pl.ds(off, GB * BC)], dstv, sem)
        pltpu.async_copy(eh.at[pl.ds(off, GB * BC)], ev, sem)
        pltpu.make_async_copy(srch.at[pl.ds(off, GB * BC)], srcv, sem).wait()
        pltpu.make_async_copy(dsth.at[pl.ds(off, GB * BC)], dstv, sem).wait()
        pltpu.make_async_copy(eh.at[pl.ds(off, GB * BC)], ev, sem).wait()
        def fill_sidx(q, b):
            for g in range(BC // 16):
                sidxs[b][pl.ds(g * 16, 16)] = srcv[pl.ds(q * BC + g * 16, 16)]

        fill_sidx(0, 0)
        pltpu.async_copy(h.at[sidxs[0]], rows_bufs[0], gsems[0])
        for q in range(GB):
            qo = q * BC
            b = q % 2
            rows = rows_bufs[b]
            pltpu.make_async_copy(h.at[sidxs[b]], rows, gsems[b]).wait()
            if q + 1 < GB:
                fill_sidx(q + 1, 1 - b)
                pltpu.async_copy(h.at[sidxs[1 - b]], rows_bufs[1 - b],
                                 gsems[1 - b])
            for g in range(BC // 16):
                uv[pl.ds(g * 16, 16)] = (
                    jnp.exp(ev[pl.ds(qo + g * 16, 16)] - M) * invS)

            def rowscale(bi, _2):
                ub = plsc.load_gather(uv, [jnp.full((16,), bi, jnp.int32)])
                for j in range(D // 16):
                    rows[bi, pl.ds(j * 16, 16)] = (
                        rows[bi, pl.ds(j * 16, 16)] * ub)
                return 0

            lax.fori_loop(0, BC, rowscale, 0)
            for g in range(BC // 16):
                didx[pl.ds(g * 16, 16)] = dstv[pl.ds(qo + g * 16, 16)]
            pltpu.sync_copy(rows, acc.at[didx], add=True)
        return 0

    lax.fori_loop(0, NCHUNK // GB, superchunk, 0)
    plsc.subcore_barrier()

    for k in range(RPT // ZROWS):
        r0 = s * RPT + k * ZROWS
        pltpu.sync_copy(acc.at[pl.ds(r0, ZROWS)], part.at[c, pl.ds(r0, ZROWS)])


def _scatter(h, src, dst, e, ms, ss):
    mesh = plsc.VectorSubcoreMesh(core_axis_name="c", subcore_axis_name="s")
    fn = pl.kernel(
        _scatter_body,
        out_type=jax.ShapeDtypeStruct((NC, N_PAD, D), _f32),
        mesh=mesh,
        compiler_params=pltpu.CompilerParams(needs_layout_passes=False),
        scratch_types=[
            pltpu.VMEM_SHARED((N_PAD, D), _f32),
            pltpu.VMEM((NW, 16), _f32),
            pltpu.VMEM((NW, 16), _f32),
            pltpu.VMEM((GB * BC,), jnp.int32),
            pltpu.VMEM((GB * BC,), jnp.int32),
            pltpu.VMEM((GB * BC,), _f32),
            pltpu.VMEM((BC,), _f32),
            pltpu.VMEM((BC,), jnp.int32),
            pltpu.VMEM((BC,), jnp.int32),
            pltpu.VMEM((BC,), jnp.int32),
            pltpu.VMEM((BC, D), _f32),
            pltpu.VMEM((BC, D), _f32),
            pltpu.VMEM((ZROWS, D), _f32),
            pltpu.SemaphoreType.DMA,
            pltpu.SemaphoreType.DMA,
            pltpu.SemaphoreType.DMA,
        ],
    )
    return fn(h, src, dst, e, ms, ss)


def _combine_body(p_ref, o_ref):
    o_ref[...] = jnp.maximum(p_ref[0] + p_ref[1], 0.0)


def _combine(part):
    nb = 10
    rb = N // nb
    return pl.pallas_call(
        _combine_body,
        grid=(nb,),
        in_specs=[pl.BlockSpec((NC, rb, D), lambda i: (0, i, 0))],
        out_specs=pl.BlockSpec((rb, D), lambda i: (i, 0)),
        out_shape=jax.ShapeDtypeStruct((N, D), _f32),
    )(part)


def kernel(h, edge_index, W, b, att_W, att_b):
    src = edge_index[0]
    dst = edge_index[1]
    a2rows = att_W.reshape(2, D)
    a8 = jnp.zeros((8, D), _f32).at[:2].set(a2rows)
    b2 = b.reshape(D, 1)
    attb = att_b.reshape(1, 1)

    s2 = _scores(h, W, a8, b2, attb)
    e, ms, ss = _edge_scores(s2, src, dst)
    part = _scatter(h, src, dst, e, ms, ss)
    return _combine(part)
